# baseline (speedup 1.0000x reference)
import jax
import jax.numpy as jnp
from jax.experimental import pallas as pl


def _relu_kernel(x_ref, o_ref):
    o_ref[...] = jnp.maximum(x_ref[...], 0.0)


def _gcn(x, src, dst, W, b, n):
    deg = jax.ops.segment_sum(jnp.ones_like(dst, dtype=x.dtype), dst, num_segments=n)
    dinv = jnp.where(deg > 0, deg ** -0.5, 0.0)
    norm = dinv[src] * dinv[dst]
    h = x @ W
    msg = h[src] * norm[:, None]
    out = jax.ops.segment_sum(msg, dst, num_segments=n)
    return out + b


def kernel(x, edge_index, W1, b1, W2, b2, Wh, bh):
    n = x.shape[0]
    self_loops = jnp.arange(n, dtype=edge_index.dtype)
    src = jnp.concatenate([edge_index[0], self_loops])
    dst = jnp.concatenate([edge_index[1], self_loops])
    h = _gcn(x, src, dst, W1, b1, n)
    h = pl.pallas_call(
        _relu_kernel,
        out_shape=jax.ShapeDtypeStruct(h.shape, h.dtype),
    )(h)
    h = _gcn(h, src, dst, W2, b2, n)
    h = pl.pallas_call(
        _relu_kernel,
        out_shape=jax.ShapeDtypeStruct(h.shape, h.dtype),
    )(h)
    logits = h @ Wh + bh
    u = jax.random.uniform(jax.random.key(42), logits.shape, dtype=logits.dtype)
    g = -jnp.log(-jnp.log(u + 1e-20) + 1e-20)
    action = jnp.argmax(logits + g, axis=-1)
    logp = jax.nn.log_softmax(logits, axis=-1)
    log_prob = jnp.take_along_axis(logp, action[:, None], axis=1).squeeze(1).sum()
    return action, log_prob


# speedup vs baseline: 18.3443x; 18.3443x over previous
import functools

import jax
import jax.numpy as jnp
from jax import lax
from jax.experimental import pallas as pl
from jax.experimental.pallas import tpu as pltpu
from jax.experimental.pallas import tpu_sc as plsc

_NC = 2
_NS = 16
_CHUNK = 100
_BM = 1000


def _sc_degree(dst3, zerosd, onesd, n_pad, d):
    _, rows_pw, chunk = dst3.shape
    nps = n_pad // _NS
    mesh = plsc.VectorSubcoreMesh(core_axis_name="c", subcore_axis_name="s", num_cores=_NC, num_subcores=_NS)

    @functools.partial(
        pl.kernel,
        out_type=jax.ShapeDtypeStruct((_NC, n_pad, d), jnp.float32),
        mesh=mesh,
        scratch_types=[
            pltpu.VMEM((rows_pw, chunk), jnp.int32),
            pltpu.VMEM((chunk, d), jnp.float32),
            pltpu.VMEM_SHARED((n_pad, d), jnp.float32),
        ],
    )
    def deg_kernel(dst_hbm, z_hbm, ones_hbm, out_hbm, dst_v, ones_v, acc):
        c = lax.axis_index("c")
        s = lax.axis_index("s")
        w = c * _NS + s
        pltpu.sync_copy(z_hbm, acc.at[pl.ds(s * nps, nps)])
        pltpu.sync_copy(dst_hbm.at[w], dst_v)
        pltpu.sync_copy(ones_hbm, ones_v)
        plsc.subcore_barrier()

        def body(i, carry):
            pltpu.sync_copy(ones_v, acc.at[dst_v.at[i]], add=True)
            return carry

        lax.fori_loop(0, rows_pw, body, 0)
        plsc.subcore_barrier()
        pltpu.sync_copy(acc.at[pl.ds(s * nps, nps)],
                        out_hbm.at[c, pl.ds(s * nps, nps)])

    return deg_kernel(dst3, zerosd, onesd)


def _sc_aggregate(hp, src3, dst3, zerosd, n_pad, d):
    _, rows_pw, chunk = src3.shape
    nps = n_pad // _NS
    mesh = plsc.VectorSubcoreMesh(core_axis_name="c", subcore_axis_name="s", num_cores=_NC, num_subcores=_NS)

    @functools.partial(
        pl.kernel,
        out_type=jax.ShapeDtypeStruct((_NC, n_pad, d), jnp.float32),
        mesh=mesh,
        scratch_types=[
            pltpu.VMEM((rows_pw, chunk), jnp.int32),
            pltpu.VMEM((rows_pw, chunk), jnp.int32),
            pltpu.VMEM((chunk, d), jnp.float32),
            pltpu.VMEM_SHARED((n_pad, d), jnp.float32),
            pltpu.SemaphoreType.DMA,
        ],
    )
    def agg_kernel(hp_hbm, src_hbm, dst_hbm, z_hbm, out_hbm,
                   src_v, dst_v, buf, acc, sem):
        c = lax.axis_index("c")
        s = lax.axis_index("s")
        w = c * _NS + s
        pltpu.sync_copy(z_hbm, acc.at[pl.ds(s * nps, nps)])
        pltpu.sync_copy(src_hbm.at[w], src_v)
        pltpu.sync_copy(dst_hbm.at[w], dst_v)
        plsc.subcore_barrier()

        def body(i, carry):
            pltpu.async_copy(hp_hbm.at[src_v.at[i]], buf, sem).wait()
            pltpu.sync_copy(buf, acc.at[dst_v.at[i]], add=True)
            return carry

        lax.fori_loop(0, rows_pw, body, 0)
        plsc.subcore_barrier()
        pltpu.sync_copy(acc.at[pl.ds(s * nps, nps)],
                        out_hbm.at[c, pl.ds(s * nps, nps)])

    return agg_kernel(hp, src3, dst3, zerosd)


def _tc_layer1(x, W1, degp):
    n, d = x.shape
    grid = n // _BM

    def body(x_ref, w_ref, dp_ref, h_ref, hp_ref, di_ref):
        h = jnp.dot(x_ref[...], w_ref[...], preferred_element_type=jnp.float32)
        deg = dp_ref[0, :, 0:1] + dp_ref[1, :, 0:1] + 1.0
        dinv = lax.rsqrt(deg)
        h_ref[...] = h
        hp_ref[...] = h * dinv
        di_ref[...] = dinv

    return pl.pallas_call(
        body,
        grid=(grid,),
        in_specs=[
            pl.BlockSpec((_BM, d), lambda i: (i, 0)),
            pl.BlockSpec((d, d), lambda i: (0, 0)),
            pl.BlockSpec((_NC, _BM, d), lambda i: (0, i, 0)),
        ],
        out_specs=[
            pl.BlockSpec((_BM, d), lambda i: (i, 0)),
            pl.BlockSpec((_BM, d), lambda i: (i, 0)),
            pl.BlockSpec((_BM, 1), lambda i: (i, 0)),
        ],
        out_shape=[
            jax.ShapeDtypeStruct((n, d), jnp.float32),
            jax.ShapeDtypeStruct((n, d), jnp.float32),
            jax.ShapeDtypeStruct((n, 1), jnp.float32),
        ],
    )(x, W1, degp)


def _tc_layer2(aggp, H1, dinv, b1, W2):
    n, d = H1.shape
    grid = n // _BM

    def body(a_ref, h_ref, di_ref, b_ref, w_ref, h2_ref, h2p_ref):
        di = di_ref[...]
        h1 = di * (a_ref[0] + a_ref[1]) + di * di * h_ref[...] + b_ref[...]
        h1 = jnp.maximum(h1, 0.0)
        h2 = jnp.dot(h1, w_ref[...], preferred_element_type=jnp.float32)
        h2_ref[...] = h2
        h2p_ref[...] = h2 * di

    return pl.pallas_call(
        body,
        grid=(grid,),
        in_specs=[
            pl.BlockSpec((_NC, _BM, d), lambda i: (0, i, 0)),
            pl.BlockSpec((_BM, d), lambda i: (i, 0)),
            pl.BlockSpec((_BM, 1), lambda i: (i, 0)),
            pl.BlockSpec((1, d), lambda i: (0, 0)),
            pl.BlockSpec((d, d), lambda i: (0, 0)),
        ],
        out_specs=[
            pl.BlockSpec((_BM, d), lambda i: (i, 0)),
            pl.BlockSpec((_BM, d), lambda i: (i, 0)),
        ],
        out_shape=[
            jax.ShapeDtypeStruct((n, d), jnp.float32),
            jax.ShapeDtypeStruct((n, d), jnp.float32),
        ],
    )(aggp, H1, dinv, b1, W2)


def _tc_head(aggp, H2, dinv, b2, Wh, bh, u):
    n, d = H2.shape
    a = Wh.shape[1]
    grid = n // _BM

    def body(ag_ref, h_ref, di_ref, b_ref, wh_ref, bh_ref, u_ref,
             act_ref, lp_ref):
        i = pl.program_id(0)
        di = di_ref[...]
        h2 = di * (ag_ref[0] + ag_ref[1]) + di * di * h_ref[...] + b_ref[...]
        h2 = jnp.maximum(h2, 0.0)
        logits = jnp.dot(h2, wh_ref[...], preferred_element_type=jnp.float32)
        logits = logits + bh_ref[...]
        uu = u_ref[...]
        g = -jnp.log(-jnp.log(uu + 1e-20) + 1e-20)
        z = logits + g
        zmax = jnp.max(z, axis=-1, keepdims=True)
        iota = lax.broadcasted_iota(jnp.int32, z.shape, 1)
        act = jnp.min(jnp.where(z >= zmax, iota, a), axis=-1)
        act_ref[...] = act[:, None]
        m = jnp.max(logits, axis=-1, keepdims=True)
        lse = m + jnp.log(jnp.sum(jnp.exp(logits - m), axis=-1, keepdims=True))
        sel = jnp.sum(jnp.where(iota == act[:, None], logits, 0.0),
                      axis=-1, keepdims=True)
        part = jnp.sum(sel - lse).reshape(1, 1)

        @pl.when(i == 0)
        def _():
            lp_ref[...] = part

        @pl.when(i != 0)
        def _():
            lp_ref[...] += part

    return pl.pallas_call(
        body,
        grid=(grid,),
        in_specs=[
            pl.BlockSpec((_NC, _BM, d), lambda i: (0, i, 0)),
            pl.BlockSpec((_BM, d), lambda i: (i, 0)),
            pl.BlockSpec((_BM, 1), lambda i: (i, 0)),
            pl.BlockSpec((1, d), lambda i: (0, 0)),
            pl.BlockSpec((d, a), lambda i: (0, 0)),
            pl.BlockSpec((1, a), lambda i: (0, 0)),
            pl.BlockSpec((_BM, a), lambda i: (i, 0)),
        ],
        out_specs=[
            pl.BlockSpec((_BM, 1), lambda i: (i, 0)),
            pl.BlockSpec((1, 1), lambda i: (0, 0)),
        ],
        out_shape=[
            jax.ShapeDtypeStruct((n, 1), jnp.int32),
            jax.ShapeDtypeStruct((1, 1), jnp.float32),
        ],
    )(aggp, H2, dinv, b2, Wh, bh, u)


def kernel(x, edge_index, W1, b1, W2, b2, Wh, bh):
    n, d = x.shape
    e = edge_index.shape[1]
    a = Wh.shape[1]
    nw = _NC * _NS
    n_pad = ((n + _NS * 8 - 1) // (_NS * 8)) * (_NS * 8)
    nps = n_pad // _NS

    src3 = edge_index[0].reshape(nw, e // _CHUNK // nw, _CHUNK)
    dst3 = edge_index[1].reshape(nw, e // _CHUNK // nw, _CHUNK)
    zerosd = jnp.zeros((nps, d), jnp.float32)
    onesd = jnp.ones((_CHUNK, d), jnp.float32)

    degp = _sc_degree(dst3, zerosd, onesd, n_pad, d)
    H1, h1p, dinv = _tc_layer1(x, W1, degp)
    aggp1 = _sc_aggregate(h1p, src3, dst3, zerosd, n_pad, d)
    H2, h2p = _tc_layer2(aggp1, H1, dinv, b1.reshape(1, d), W2)
    aggp2 = _sc_aggregate(h2p, src3, dst3, zerosd, n_pad, d)
    u = jax.random.uniform(jax.random.key(42), (n, a), dtype=jnp.float32)
    act2, lp = _tc_head(aggp2, H2, dinv, b2.reshape(1, d), Wh, bh.reshape(1, a), u)
    return act2[:, 0], lp[0, 0]


# speedup vs baseline: 20.8075x; 1.1343x over previous
import functools

import jax
import jax.numpy as jnp
from jax import lax
from jax.experimental import pallas as pl
from jax.experimental.pallas import tpu as pltpu
from jax.experimental.pallas import tpu_sc as plsc

_NC = 2
_NS = 16
_CHUNK = 100
_ACHUNK = 50
_BM = 1000


def _sc_degree(dst3, zerosd, onesd, n_pad, d):
    _, rows_pw, chunk = dst3.shape
    nps = n_pad // _NS
    mesh = plsc.VectorSubcoreMesh(core_axis_name="c", subcore_axis_name="s", num_cores=_NC, num_subcores=_NS)

    @functools.partial(
        pl.kernel,
        out_type=jax.ShapeDtypeStruct((_NC, n_pad, d), jnp.float32),
        mesh=mesh,
        scratch_types=[
            pltpu.VMEM((rows_pw, chunk), jnp.int32),
            pltpu.VMEM((chunk, d), jnp.float32),
            pltpu.VMEM_SHARED((n_pad, d), jnp.float32),
        ],
    )
    def deg_kernel(dst_hbm, z_hbm, ones_hbm, out_hbm, dst_v, ones_v, acc):
        c = lax.axis_index("c")
        s = lax.axis_index("s")
        w = c * _NS + s
        pltpu.sync_copy(z_hbm, acc.at[pl.ds(s * nps, nps)])
        pltpu.sync_copy(dst_hbm.at[w], dst_v)
        pltpu.sync_copy(ones_hbm, ones_v)
        plsc.subcore_barrier()

        def body(i, carry):
            pltpu.sync_copy(ones_v, acc.at[dst_v.at[i]], add=True)
            return carry

        lax.fori_loop(0, rows_pw, body, 0)
        plsc.subcore_barrier()
        pltpu.sync_copy(acc.at[pl.ds(s * nps, nps)],
                        out_hbm.at[c, pl.ds(s * nps, nps)])

    return deg_kernel(dst3, zerosd, onesd)


def _sc_aggregate(hp, src3, dst3, zerosd, n_pad, d):
    _, rows_pw, chunk = src3.shape
    nps = n_pad // _NS
    nb = 4
    ib = 8
    n_blocks = rows_pw // ib
    mesh = plsc.VectorSubcoreMesh(core_axis_name="c", subcore_axis_name="s", num_cores=_NC, num_subcores=_NS)

    @functools.partial(
        pl.kernel,
        out_type=jax.ShapeDtypeStruct((_NC, n_pad, d), jnp.float32),
        mesh=mesh,
        scratch_types=[
            pltpu.VMEM((ib, chunk), jnp.int32),
            pltpu.VMEM((ib, chunk), jnp.int32),
            [pltpu.VMEM((chunk, d), jnp.float32) for _ in range(nb)],
            [pltpu.SemaphoreType.DMA for _ in range(nb)],
            pltpu.VMEM_SHARED((n_pad, d), jnp.float32),
        ],
    )
    def agg_kernel(hp_hbm, src_hbm, dst_hbm, z_hbm, out_hbm,
                   src_v, dst_v, bufs, sems, acc):
        c = lax.axis_index("c")
        s = lax.axis_index("s")
        w = c * _NS + s
        pltpu.sync_copy(z_hbm, acc.at[pl.ds(s * nps, nps)])
        plsc.subcore_barrier()

        def block_body(blk, carry):
            base = blk * ib
            pltpu.sync_copy(src_hbm.at[w, pl.ds(base, ib)], src_v)
            pltpu.sync_copy(dst_hbm.at[w, pl.ds(base, ib)], dst_v)
            for b in range(nb):
                pltpu.async_copy(hp_hbm.at[src_v.at[b]], bufs[b], sems[b])
            for j in range(ib):
                b = j % nb
                pltpu.make_async_copy(hp_hbm.at[src_v.at[j]], bufs[b],
                                      sems[b]).wait()
                pltpu.sync_copy(bufs[b], acc.at[dst_v.at[j]], add=True)
                if j + nb < ib:
                    pltpu.async_copy(hp_hbm.at[src_v.at[j + nb]], bufs[b],
                                     sems[b])
            return carry

        lax.fori_loop(0, n_blocks, block_body, 0)
        plsc.subcore_barrier()
        pltpu.sync_copy(acc.at[pl.ds(s * nps, nps)],
                        out_hbm.at[c, pl.ds(s * nps, nps)])

    return agg_kernel(hp, src3, dst3, zerosd)


def _tc_layer1(x, W1, degp):
    n, d = x.shape
    grid = n // _BM

    def body(x_ref, w_ref, dp_ref, h_ref, hp_ref, di_ref):
        h = jnp.dot(x_ref[...], w_ref[...], preferred_element_type=jnp.float32)
        deg = dp_ref[0, :, 0:1] + dp_ref[1, :, 0:1] + 1.0
        dinv = lax.rsqrt(deg)
        h_ref[...] = h
        hp_ref[...] = h * dinv
        di_ref[...] = dinv

    return pl.pallas_call(
        body,
        grid=(grid,),
        in_specs=[
            pl.BlockSpec((_BM, d), lambda i: (i, 0)),
            pl.BlockSpec((d, d), lambda i: (0, 0)),
            pl.BlockSpec((_NC, _BM, d), lambda i: (0, i, 0)),
        ],
        out_specs=[
            pl.BlockSpec((_BM, d), lambda i: (i, 0)),
            pl.BlockSpec((_BM, d), lambda i: (i, 0)),
            pl.BlockSpec((_BM, 1), lambda i: (i, 0)),
        ],
        out_shape=[
            jax.ShapeDtypeStruct((n, d), jnp.float32),
            jax.ShapeDtypeStruct((n, d), jnp.float32),
            jax.ShapeDtypeStruct((n, 1), jnp.float32),
        ],
    )(x, W1, degp)


def _tc_layer2(aggp, H1, dinv, b1, W2):
    n, d = H1.shape
    grid = n // _BM

    def body(a_ref, h_ref, di_ref, b_ref, w_ref, h2_ref, h2p_ref):
        di = di_ref[...]
        h1 = di * (a_ref[0] + a_ref[1]) + di * di * h_ref[...] + b_ref[...]
        h1 = jnp.maximum(h1, 0.0)
        h2 = jnp.dot(h1, w_ref[...], preferred_element_type=jnp.float32)
        h2_ref[...] = h2
        h2p_ref[...] = h2 * di

    return pl.pallas_call(
        body,
        grid=(grid,),
        in_specs=[
            pl.BlockSpec((_NC, _BM, d), lambda i: (0, i, 0)),
            pl.BlockSpec((_BM, d), lambda i: (i, 0)),
            pl.BlockSpec((_BM, 1), lambda i: (i, 0)),
            pl.BlockSpec((1, d), lambda i: (0, 0)),
            pl.BlockSpec((d, d), lambda i: (0, 0)),
        ],
        out_specs=[
            pl.BlockSpec((_BM, d), lambda i: (i, 0)),
            pl.BlockSpec((_BM, d), lambda i: (i, 0)),
        ],
        out_shape=[
            jax.ShapeDtypeStruct((n, d), jnp.float32),
            jax.ShapeDtypeStruct((n, d), jnp.float32),
        ],
    )(aggp, H1, dinv, b1, W2)


def _tc_head(aggp, H2, dinv, b2, Wh, bh, u):
    n, d = H2.shape
    a = Wh.shape[1]
    grid = n // _BM

    def body(ag_ref, h_ref, di_ref, b_ref, wh_ref, bh_ref, u_ref,
             act_ref, lp_ref):
        i = pl.program_id(0)
        di = di_ref[...]
        h2 = di * (ag_ref[0] + ag_ref[1]) + di * di * h_ref[...] + b_ref[...]
        h2 = jnp.maximum(h2, 0.0)
        logits = jnp.dot(h2, wh_ref[...], preferred_element_type=jnp.float32)
        logits = logits + bh_ref[...]
        uu = u_ref[...]
        g = -jnp.log(-jnp.log(uu + 1e-20) + 1e-20)
        z = logits + g
        zmax = jnp.max(z, axis=-1, keepdims=True)
        iota = lax.broadcasted_iota(jnp.int32, z.shape, 1)
        act = jnp.min(jnp.where(z >= zmax, iota, a), axis=-1)
        act_ref[...] = act[:, None]
        m = jnp.max(logits, axis=-1, keepdims=True)
        lse = m + jnp.log(jnp.sum(jnp.exp(logits - m), axis=-1, keepdims=True))
        sel = jnp.sum(jnp.where(iota == act[:, None], logits, 0.0),
                      axis=-1, keepdims=True)
        part = jnp.sum(sel - lse).reshape(1, 1)

        @pl.when(i == 0)
        def _():
            lp_ref[...] = part

        @pl.when(i != 0)
        def _():
            lp_ref[...] += part

    return pl.pallas_call(
        body,
        grid=(grid,),
        in_specs=[
            pl.BlockSpec((_NC, _BM, d), lambda i: (0, i, 0)),
            pl.BlockSpec((_BM, d), lambda i: (i, 0)),
            pl.BlockSpec((_BM, 1), lambda i: (i, 0)),
            pl.BlockSpec((1, d), lambda i: (0, 0)),
            pl.BlockSpec((d, a), lambda i: (0, 0)),
            pl.BlockSpec((1, a), lambda i: (0, 0)),
            pl.BlockSpec((_BM, a), lambda i: (i, 0)),
        ],
        out_specs=[
            pl.BlockSpec((_BM, 1), lambda i: (i, 0)),
            pl.BlockSpec((1, 1), lambda i: (0, 0)),
        ],
        out_shape=[
            jax.ShapeDtypeStruct((n, 1), jnp.int32),
            jax.ShapeDtypeStruct((1, 1), jnp.float32),
        ],
    )(aggp, H2, dinv, b2, Wh, bh, u)


def kernel(x, edge_index, W1, b1, W2, b2, Wh, bh):
    n, d = x.shape
    e = edge_index.shape[1]
    a = Wh.shape[1]
    nw = _NC * _NS
    n_pad = ((n + _NS * 8 - 1) // (_NS * 8)) * (_NS * 8)
    nps = n_pad // _NS

    src3 = edge_index[0].reshape(nw, e // _ACHUNK // nw, _ACHUNK)
    dst3 = edge_index[1].reshape(nw, e // _ACHUNK // nw, _ACHUNK)
    dst3d = edge_index[1].reshape(nw, e // _CHUNK // nw, _CHUNK)
    zerosd = jnp.zeros((nps, d), jnp.float32)
    onesd = jnp.ones((_CHUNK, d), jnp.float32)

    degp = _sc_degree(dst3d, zerosd, onesd, n_pad, d)
    H1, h1p, dinv = _tc_layer1(x, W1, degp)
    aggp1 = _sc_aggregate(h1p, src3, dst3, zerosd, n_pad, d)
    H2, h2p = _tc_layer2(aggp1, H1, dinv, b1.reshape(1, d), W2)
    aggp2 = _sc_aggregate(h2p, src3, dst3, zerosd, n_pad, d)
    u = jax.random.uniform(jax.random.key(42), (n, a), dtype=jnp.float32)
    act2, lp = _tc_head(aggp2, H2, dinv, b2.reshape(1, d), Wh, bh.reshape(1, a), u)
    return act2[:, 0], lp[0, 0]


# speedup vs baseline: 20.8673x; 1.0029x over previous
import functools

import jax
import jax.numpy as jnp
from jax import lax
from jax.experimental import pallas as pl
from jax.experimental.pallas import tpu as pltpu
from jax.experimental.pallas import tpu_sc as plsc

_NC = 2
_NS = 16
_CHUNK = 100
_ACHUNK = 50
_BM = 1000


def _sc_degree(dst3, zerosd, onesd, n_pad, d):
    _, rows_pw, chunk = dst3.shape
    nps = n_pad // _NS
    fk = 4
    mesh = plsc.VectorSubcoreMesh(core_axis_name="c", subcore_axis_name="s", num_cores=_NC, num_subcores=_NS)

    @functools.partial(
        pl.kernel,
        out_type=jax.ShapeDtypeStruct((_NC, n_pad, d), jnp.float32),
        mesh=mesh,
        scratch_types=[
            pltpu.VMEM((rows_pw, chunk), jnp.int32),
            pltpu.VMEM((chunk, d), jnp.float32),
            pltpu.SemaphoreType.DMA,
            pltpu.VMEM_SHARED((n_pad, d), jnp.float32),
        ],
    )
    def deg_kernel(dst_hbm, z_hbm, ones_hbm, out_hbm, dst_v, ones_v, sem, acc):
        c = lax.axis_index("c")
        s = lax.axis_index("s")
        w = c * _NS + s
        pltpu.sync_copy(z_hbm, acc.at[pl.ds(s * nps, nps)])
        pltpu.sync_copy(dst_hbm.at[w], dst_v)
        pltpu.sync_copy(ones_hbm, ones_v)
        plsc.subcore_barrier()

        def round_body(r, carry):
            for k in range(fk):
                pltpu.make_async_copy(
                    ones_v, acc.at[dst_v.at[r * fk + k]], sem).start(add=True)
            for k in range(fk):
                pltpu.make_async_copy(
                    ones_v, acc.at[dst_v.at[r * fk + k]], sem).wait()
            return carry

        lax.fori_loop(0, rows_pw // fk, round_body, 0)
        plsc.subcore_barrier()
        pltpu.sync_copy(acc.at[pl.ds(s * nps, nps)],
                        out_hbm.at[c, pl.ds(s * nps, nps)])

    return deg_kernel(dst3, zerosd, onesd)


def _sc_aggregate(hp, src3, dst3, zerosd, n_pad, d):
    _, rows_pw, chunk = src3.shape
    nps = n_pad // _NS
    nb = 4
    ib = 8
    n_blocks = rows_pw // ib
    mesh = plsc.VectorSubcoreMesh(core_axis_name="c", subcore_axis_name="s", num_cores=_NC, num_subcores=_NS)

    @functools.partial(
        pl.kernel,
        out_type=jax.ShapeDtypeStruct((_NC, n_pad, d), jnp.float32),
        mesh=mesh,
        scratch_types=[
            pltpu.VMEM((ib, chunk), jnp.int32),
            pltpu.VMEM((ib, chunk), jnp.int32),
            [pltpu.VMEM((chunk, d), jnp.float32) for _ in range(nb)],
            [pltpu.SemaphoreType.DMA for _ in range(nb)],
            pltpu.VMEM_SHARED((n_pad, d), jnp.float32),
        ],
    )
    def agg_kernel(hp_hbm, src_hbm, dst_hbm, z_hbm, out_hbm,
                   src_v, dst_v, bufs, sems, acc):
        c = lax.axis_index("c")
        s = lax.axis_index("s")
        w = c * _NS + s
        pltpu.sync_copy(z_hbm, acc.at[pl.ds(s * nps, nps)])
        plsc.subcore_barrier()

        def block_body(blk, carry):
            base = blk * ib
            pltpu.sync_copy(src_hbm.at[w, pl.ds(base, ib)], src_v)
            pltpu.sync_copy(dst_hbm.at[w, pl.ds(base, ib)], dst_v)
            for b in range(nb):
                pltpu.async_copy(hp_hbm.at[src_v.at[b]], bufs[b], sems[b])
            for j in range(ib):
                b = j % nb
                pltpu.make_async_copy(hp_hbm.at[src_v.at[j]], bufs[b],
                                      sems[b]).wait()
                pltpu.sync_copy(bufs[b], acc.at[dst_v.at[j]], add=True)
                if j + nb < ib:
                    pltpu.async_copy(hp_hbm.at[src_v.at[j + nb]], bufs[b],
                                     sems[b])
            return carry

        lax.fori_loop(0, n_blocks, block_body, 0)
        plsc.subcore_barrier()
        pltpu.sync_copy(acc.at[pl.ds(s * nps, nps)],
                        out_hbm.at[c, pl.ds(s * nps, nps)])

    return agg_kernel(hp, src3, dst3, zerosd)


def _tc_layer1(x, W1, degcol):
    n, d = x.shape
    grid = n // _BM

    def body(x_ref, w_ref, dp_ref, h_ref, hp_ref, di_ref):
        h = jnp.dot(x_ref[...], w_ref[...], preferred_element_type=jnp.float32)
        deg = dp_ref[0, :, 0:1] + dp_ref[1, :, 0:1] + 1.0
        dinv = lax.rsqrt(deg)
        h_ref[...] = h
        hp_ref[...] = h * dinv
        di_ref[...] = dinv

    return pl.pallas_call(
        body,
        grid=(grid,),
        in_specs=[
            pl.BlockSpec((_BM, d), lambda i: (i, 0)),
            pl.BlockSpec((d, d), lambda i: (0, 0)),
            pl.BlockSpec((_NC, _BM, d), lambda i: (0, i, 0)),
        ],
        out_specs=[
            pl.BlockSpec((_BM, d), lambda i: (i, 0)),
            pl.BlockSpec((_BM, d), lambda i: (i, 0)),
            pl.BlockSpec((_BM, 1), lambda i: (i, 0)),
        ],
        out_shape=[
            jax.ShapeDtypeStruct((n, d), jnp.float32),
            jax.ShapeDtypeStruct((n, d), jnp.float32),
            jax.ShapeDtypeStruct((n, 1), jnp.float32),
        ],
    )(x, W1, degcol)


def _tc_layer2(aggp, H1, dinv, b1, W2):
    n, d = H1.shape
    grid = n // _BM

    def body(a_ref, h_ref, di_ref, b_ref, w_ref, h2_ref, h2p_ref):
        di = di_ref[...]
        h1 = di * (a_ref[0] + a_ref[1]) + di * di * h_ref[...] + b_ref[...]
        h1 = jnp.maximum(h1, 0.0)
        h2 = jnp.dot(h1, w_ref[...], preferred_element_type=jnp.float32)
        h2_ref[...] = h2
        h2p_ref[...] = h2 * di

    return pl.pallas_call(
        body,
        grid=(grid,),
        in_specs=[
            pl.BlockSpec((_NC, _BM, d), lambda i: (0, i, 0)),
            pl.BlockSpec((_BM, d), lambda i: (i, 0)),
            pl.BlockSpec((_BM, 1), lambda i: (i, 0)),
            pl.BlockSpec((1, d), lambda i: (0, 0)),
            pl.BlockSpec((d, d), lambda i: (0, 0)),
        ],
        out_specs=[
            pl.BlockSpec((_BM, d), lambda i: (i, 0)),
            pl.BlockSpec((_BM, d), lambda i: (i, 0)),
        ],
        out_shape=[
            jax.ShapeDtypeStruct((n, d), jnp.float32),
            jax.ShapeDtypeStruct((n, d), jnp.float32),
        ],
    )(aggp, H1, dinv, b1, W2)


def _tc_head(aggp, H2, dinv, b2, Wh, bh, u):
    n, d = H2.shape
    a = Wh.shape[1]
    grid = n // _BM

    def body(ag_ref, h_ref, di_ref, b_ref, wh_ref, bh_ref, u_ref,
             act_ref, lp_ref):
        i = pl.program_id(0)
        di = di_ref[...]
        h2 = di * (ag_ref[0] + ag_ref[1]) + di * di * h_ref[...] + b_ref[...]
        h2 = jnp.maximum(h2, 0.0)
        logits = jnp.dot(h2, wh_ref[...], preferred_element_type=jnp.float32)
        logits = logits + bh_ref[...]
        uu = u_ref[...]
        g = -jnp.log(-jnp.log(uu + 1e-20) + 1e-20)
        z = logits + g
        zmax = jnp.max(z, axis=-1, keepdims=True)
        iota = lax.broadcasted_iota(jnp.int32, z.shape, 1)
        act = jnp.min(jnp.where(z >= zmax, iota, a), axis=-1)
        act_ref[...] = act[:, None]
        m = jnp.max(logits, axis=-1, keepdims=True)
        lse = m + jnp.log(jnp.sum(jnp.exp(logits - m), axis=-1, keepdims=True))
        sel = jnp.sum(jnp.where(iota == act[:, None], logits, 0.0),
                      axis=-1, keepdims=True)
        part = jnp.sum(sel - lse).reshape(1, 1)

        @pl.when(i == 0)
        def _():
            lp_ref[...] = part

        @pl.when(i != 0)
        def _():
            lp_ref[...] += part

    return pl.pallas_call(
        body,
        grid=(grid,),
        in_specs=[
            pl.BlockSpec((_NC, _BM, d), lambda i: (0, i, 0)),
            pl.BlockSpec((_BM, d), lambda i: (i, 0)),
            pl.BlockSpec((_BM, 1), lambda i: (i, 0)),
            pl.BlockSpec((1, d), lambda i: (0, 0)),
            pl.BlockSpec((d, a), lambda i: (0, 0)),
            pl.BlockSpec((1, a), lambda i: (0, 0)),
            pl.BlockSpec((_BM, a), lambda i: (i, 0)),
        ],
        out_specs=[
            pl.BlockSpec((_BM, 1), lambda i: (i, 0)),
            pl.BlockSpec((1, 1), lambda i: (0, 0)),
        ],
        out_shape=[
            jax.ShapeDtypeStruct((n, 1), jnp.int32),
            jax.ShapeDtypeStruct((1, 1), jnp.float32),
        ],
    )(aggp, H2, dinv, b2, Wh, bh, u)


def kernel(x, edge_index, W1, b1, W2, b2, Wh, bh):
    n, d = x.shape
    e = edge_index.shape[1]
    a = Wh.shape[1]
    nw = _NC * _NS
    n_pad = ((n + _NS * 8 - 1) // (_NS * 8)) * (_NS * 8)
    nps = n_pad // _NS

    src3 = edge_index[0].reshape(nw, e // _ACHUNK // nw, _ACHUNK)
    dst3 = edge_index[1].reshape(nw, e // _ACHUNK // nw, _ACHUNK)
    dst3d = edge_index[1].reshape(nw, e // _CHUNK // nw, _CHUNK)
    zerosd = jnp.zeros((nps, d), jnp.float32)
    onesd = jnp.ones((_CHUNK, d), jnp.float32)

    degp = _sc_degree(dst3d, zerosd, onesd, n_pad, d)
    H1, h1p, dinv = _tc_layer1(x, W1, degp)
    aggp1 = _sc_aggregate(h1p, src3, dst3, zerosd, n_pad, d)
    H2, h2p = _tc_layer2(aggp1, H1, dinv, b1.reshape(1, d), W2)
    aggp2 = _sc_aggregate(h2p, src3, dst3, zerosd, n_pad, d)
    u = jax.random.uniform(jax.random.key(42), (n, a), dtype=jnp.float32)
    act2, lp = _tc_head(aggp2, H2, dinv, b2.reshape(1, d), Wh, bh.reshape(1, a), u)
    return act2[:, 0], lp[0, 0]


# speedup vs baseline: 21.9024x; 1.0496x over previous
import functools

import jax
import jax.numpy as jnp
from jax import lax
from jax.experimental import pallas as pl
from jax.experimental.pallas import tpu as pltpu
from jax.experimental.pallas import tpu_sc as plsc

_NC = 2
_NS = 16
_CHUNK = 100
_ACHUNK = 50
_BM = 1000


def _sc_degree(dst3, zerosd, onesd, n_pad, d):
    _, rows_pw, chunk = dst3.shape
    nps = n_pad // _NS
    fk = 4
    mesh = plsc.VectorSubcoreMesh(core_axis_name="c", subcore_axis_name="s", num_cores=_NC, num_subcores=_NS)

    @functools.partial(
        pl.kernel,
        out_type=jax.ShapeDtypeStruct((_NC, n_pad, d), jnp.float32),
        mesh=mesh,
        scratch_types=[
            pltpu.VMEM((rows_pw, chunk), jnp.int32),
            pltpu.VMEM((chunk, d), jnp.float32),
            pltpu.SemaphoreType.DMA,
            pltpu.VMEM_SHARED((n_pad, d), jnp.float32),
        ],
    )
    def deg_kernel(dst_hbm, z_hbm, ones_hbm, out_hbm, dst_v, ones_v, sem, acc):
        c = lax.axis_index("c")
        s = lax.axis_index("s")
        w = c * _NS + s
        pltpu.sync_copy(z_hbm, acc.at[pl.ds(s * nps, nps)])
        pltpu.sync_copy(dst_hbm.at[w], dst_v)
        pltpu.sync_copy(ones_hbm, ones_v)
        plsc.subcore_barrier()

        def round_body(r, carry):
            for k in range(fk):
                pltpu.make_async_copy(
                    ones_v, acc.at[dst_v.at[r * fk + k]], sem).start(add=True)
            for k in range(fk):
                pltpu.make_async_copy(
                    ones_v, acc.at[dst_v.at[r * fk + k]], sem).wait()
            return carry

        lax.fori_loop(0, rows_pw // fk, round_body, 0)
        plsc.subcore_barrier()
        pltpu.sync_copy(acc.at[pl.ds(s * nps, nps)],
                        out_hbm.at[c, pl.ds(s * nps, nps)])

    return deg_kernel(dst3, zerosd, onesd)


def _sc_aggregate(hp, src4, dst4, zerosd, n_pad, d):
    _, nblk, ib, chunk = src4.shape
    nps = n_pad // _NS
    nb = 4
    mesh = plsc.VectorSubcoreMesh(core_axis_name="c", subcore_axis_name="s", num_cores=_NC, num_subcores=_NS)

    @functools.partial(
        pl.kernel,
        out_type=jax.ShapeDtypeStruct((_NC, n_pad, d), jnp.float32),
        mesh=mesh,
        scratch_types=[
            pltpu.VMEM((ib, chunk), jnp.int32),
            pltpu.VMEM((ib, chunk), jnp.int32),
            [pltpu.VMEM((chunk, d), jnp.float32) for _ in range(nb)],
            [pltpu.SemaphoreType.DMA for _ in range(nb)],
            [pltpu.SemaphoreType.DMA for _ in range(nb)],
            pltpu.VMEM_SHARED((n_pad, d), jnp.float32),
        ],
    )
    def agg_kernel(hp_hbm, src_hbm, dst_hbm, z_hbm, out_hbm,
                   src_v, dst_v, bufs, gsem, ssem, acc):
        c = lax.axis_index("c")
        s = lax.axis_index("s")
        w = c * _NS + s
        pltpu.sync_copy(z_hbm, acc.at[pl.ds(s * nps, nps)])
        plsc.subcore_barrier()

        def block_body(blk, carry):
            pltpu.sync_copy(src_hbm.at[w, blk], src_v)
            pltpu.sync_copy(dst_hbm.at[w, blk], dst_v)
            for j in range(2):
                pltpu.async_copy(hp_hbm.at[src_v.at[j]], bufs[j], gsem[j])
            for j in range(ib):
                b = j % nb
                pltpu.make_async_copy(hp_hbm.at[src_v.at[j]], bufs[b],
                                      gsem[b]).wait()
                pltpu.make_async_copy(bufs[b], acc.at[dst_v.at[j]],
                                      ssem[b]).start(add=True)
                if j + 2 < ib:
                    b2 = (j + 2) % nb
                    if j >= 2:
                        pltpu.make_async_copy(bufs[b2], acc.at[dst_v.at[0]],
                                              ssem[b2]).wait()
                    pltpu.async_copy(hp_hbm.at[src_v.at[j + 2]], bufs[b2],
                                     gsem[b2])
            for j in range(ib - 2, ib):
                b = j % nb
                pltpu.make_async_copy(bufs[b], acc.at[dst_v.at[0]],
                                      ssem[b]).wait()
            return carry

        lax.fori_loop(0, nblk, block_body, 0)
        plsc.subcore_barrier()
        pltpu.sync_copy(acc.at[pl.ds(s * nps, nps)],
                        out_hbm.at[c, pl.ds(s * nps, nps)])

    return agg_kernel(hp, src4, dst4, zerosd)


def _tc_layer1(x, W1, degcol):
    n, d = x.shape
    grid = n // _BM

    def body(x_ref, w_ref, dp_ref, h_ref, hp_ref, di_ref):
        h = jnp.dot(x_ref[...], w_ref[...], preferred_element_type=jnp.float32)
        deg = dp_ref[0, :, 0:1] + dp_ref[1, :, 0:1] + 1.0
        dinv = lax.rsqrt(deg)
        h_ref[...] = h
        hp_ref[...] = h * dinv
        di_ref[...] = dinv

    return pl.pallas_call(
        body,
        grid=(grid,),
        in_specs=[
            pl.BlockSpec((_BM, d), lambda i: (i, 0)),
            pl.BlockSpec((d, d), lambda i: (0, 0)),
            pl.BlockSpec((_NC, _BM, d), lambda i: (0, i, 0)),
        ],
        out_specs=[
            pl.BlockSpec((_BM, d), lambda i: (i, 0)),
            pl.BlockSpec((_BM, d), lambda i: (i, 0)),
            pl.BlockSpec((_BM, 1), lambda i: (i, 0)),
        ],
        out_shape=[
            jax.ShapeDtypeStruct((n, d), jnp.float32),
            jax.ShapeDtypeStruct((n, d), jnp.float32),
            jax.ShapeDtypeStruct((n, 1), jnp.float32),
        ],
    )(x, W1, degcol)


def _tc_layer2(aggp, H1, dinv, b1, W2):
    n, d = H1.shape
    grid = n // _BM

    def body(a_ref, h_ref, di_ref, b_ref, w_ref, h2_ref, h2p_ref):
        di = di_ref[...]
        h1 = di * (a_ref[0] + a_ref[1]) + di * di * h_ref[...] + b_ref[...]
        h1 = jnp.maximum(h1, 0.0)
        h2 = jnp.dot(h1, w_ref[...], preferred_element_type=jnp.float32)
        h2_ref[...] = h2
        h2p_ref[...] = h2 * di

    return pl.pallas_call(
        body,
        grid=(grid,),
        in_specs=[
            pl.BlockSpec((_NC, _BM, d), lambda i: (0, i, 0)),
            pl.BlockSpec((_BM, d), lambda i: (i, 0)),
            pl.BlockSpec((_BM, 1), lambda i: (i, 0)),
            pl.BlockSpec((1, d), lambda i: (0, 0)),
            pl.BlockSpec((d, d), lambda i: (0, 0)),
        ],
        out_specs=[
            pl.BlockSpec((_BM, d), lambda i: (i, 0)),
            pl.BlockSpec((_BM, d), lambda i: (i, 0)),
        ],
        out_shape=[
            jax.ShapeDtypeStruct((n, d), jnp.float32),
            jax.ShapeDtypeStruct((n, d), jnp.float32),
        ],
    )(aggp, H1, dinv, b1, W2)


def _tc_head(aggp, H2, dinv, b2, Wh, bh, u):
    n, d = H2.shape
    a = Wh.shape[1]
    grid = n // _BM

    def body(ag_ref, h_ref, di_ref, b_ref, wh_ref, bh_ref, u_ref,
             act_ref, lp_ref):
        i = pl.program_id(0)
        di = di_ref[...]
        h2 = di * (ag_ref[0] + ag_ref[1]) + di * di * h_ref[...] + b_ref[...]
        h2 = jnp.maximum(h2, 0.0)
        logits = jnp.dot(h2, wh_ref[...], preferred_element_type=jnp.float32)
        logits = logits + bh_ref[...]
        uu = u_ref[...]
        g = -jnp.log(-jnp.log(uu + 1e-20) + 1e-20)
        z = logits + g
        zmax = jnp.max(z, axis=-1, keepdims=True)
        iota = lax.broadcasted_iota(jnp.int32, z.shape, 1)
        act = jnp.min(jnp.where(z >= zmax, iota, a), axis=-1)
        act_ref[...] = act[:, None]
        m = jnp.max(logits, axis=-1, keepdims=True)
        lse = m + jnp.log(jnp.sum(jnp.exp(logits - m), axis=-1, keepdims=True))
        sel = jnp.sum(jnp.where(iota == act[:, None], logits, 0.0),
                      axis=-1, keepdims=True)
        part = jnp.sum(sel - lse).reshape(1, 1)

        @pl.when(i == 0)
        def _():
            lp_ref[...] = part

        @pl.when(i != 0)
        def _():
            lp_ref[...] += part

    return pl.pallas_call(
        body,
        grid=(grid,),
        in_specs=[
            pl.BlockSpec((_NC, _BM, d), lambda i: (0, i, 0)),
            pl.BlockSpec((_BM, d), lambda i: (i, 0)),
            pl.BlockSpec((_BM, 1), lambda i: (i, 0)),
            pl.BlockSpec((1, d), lambda i: (0, 0)),
            pl.BlockSpec((d, a), lambda i: (0, 0)),
            pl.BlockSpec((1, a), lambda i: (0, 0)),
            pl.BlockSpec((_BM, a), lambda i: (i, 0)),
        ],
        out_specs=[
            pl.BlockSpec((_BM, 1), lambda i: (i, 0)),
            pl.BlockSpec((1, 1), lambda i: (0, 0)),
        ],
        out_shape=[
            jax.ShapeDtypeStruct((n, 1), jnp.int32),
            jax.ShapeDtypeStruct((1, 1), jnp.float32),
        ],
    )(aggp, H2, dinv, b2, Wh, bh, u)


def kernel(x, edge_index, W1, b1, W2, b2, Wh, bh):
    n, d = x.shape
    e = edge_index.shape[1]
    a = Wh.shape[1]
    nw = _NC * _NS
    n_pad = ((n + _NS * 8 - 1) // (_NS * 8)) * (_NS * 8)
    nps = n_pad // _NS

    nblk = e // (nw * 20 * _ACHUNK)
    src4 = edge_index[0].reshape(nw, nblk, 20, _ACHUNK)
    dst4 = edge_index[1].reshape(nw, nblk, 20, _ACHUNK)
    dst3d = edge_index[1].reshape(nw, e // _CHUNK // nw, _CHUNK)
    zerosd = jnp.zeros((nps, d), jnp.float32)
    onesd = jnp.ones((_CHUNK, d), jnp.float32)

    degp = _sc_degree(dst3d, zerosd, onesd, n_pad, d)
    H1, h1p, dinv = _tc_layer1(x, W1, degp)
    aggp1 = _sc_aggregate(h1p, src4, dst4, zerosd, n_pad, d)
    H2, h2p = _tc_layer2(aggp1, H1, dinv, b1.reshape(1, d), W2)
    aggp2 = _sc_aggregate(h2p, src4, dst4, zerosd, n_pad, d)
    u = jax.random.uniform(jax.random.key(42), (n, a), dtype=jnp.float32)
    act2, lp = _tc_head(aggp2, H2, dinv, b2.reshape(1, d), Wh, bh.reshape(1, a), u)
    return act2[:, 0], lp[0, 0]


# speedup vs baseline: 22.6812x; 1.0356x over previous
import functools

import jax
import jax.numpy as jnp
from jax import lax
from jax.experimental import pallas as pl
from jax.experimental.pallas import tpu as pltpu
from jax.experimental.pallas import tpu_sc as plsc

_NC = 2
_NS = 16
_CHUNK = 100
_ACHUNK = 50
_BM = 1000


def _sc_degree(dst3, zerosd, onesd, n_pad, d):
    _, rows_pw, chunk = dst3.shape
    nps = n_pad // _NS
    fk = 4
    mesh = plsc.VectorSubcoreMesh(core_axis_name="c", subcore_axis_name="s", num_cores=_NC, num_subcores=_NS)

    @functools.partial(
        pl.kernel,
        out_type=jax.ShapeDtypeStruct((_NC, n_pad, d), jnp.float32),
        mesh=mesh,
        scratch_types=[
            pltpu.VMEM((rows_pw, chunk), jnp.int32),
            pltpu.VMEM((chunk, d), jnp.float32),
            pltpu.SemaphoreType.DMA,
            pltpu.VMEM_SHARED((n_pad, d), jnp.float32),
        ],
    )
    def deg_kernel(dst_hbm, z_hbm, ones_hbm, out_hbm, dst_v, ones_v, sem, acc):
        c = lax.axis_index("c")
        s = lax.axis_index("s")
        w = c * _NS + s
        pltpu.sync_copy(z_hbm, acc.at[pl.ds(s * nps, nps)])
        pltpu.sync_copy(dst_hbm.at[w], dst_v)
        pltpu.sync_copy(ones_hbm, ones_v)
        plsc.subcore_barrier()

        def round_body(r, carry):
            for k in range(fk):
                pltpu.make_async_copy(
                    ones_v, acc.at[dst_v.at[r * fk + k]], sem).start(add=True)
            for k in range(fk):
                pltpu.make_async_copy(
                    ones_v, acc.at[dst_v.at[r * fk + k]], sem).wait()
            return carry

        lax.fori_loop(0, rows_pw // fk, round_body, 0)
        plsc.subcore_barrier()
        pltpu.sync_copy(acc.at[pl.ds(s * nps, nps)],
                        out_hbm.at[c, pl.ds(s * nps, nps)])

    return deg_kernel(dst3, zerosd, onesd)


def _sc_aggregate(hp, src4, dst4, zerosd, n_pad, d):
    _, nblk, ib, chunk = src4.shape
    nps = n_pad // _NS
    nb = 4
    mesh = plsc.VectorSubcoreMesh(core_axis_name="c", subcore_axis_name="s", num_cores=_NC, num_subcores=_NS)

    @functools.partial(
        pl.kernel,
        out_type=jax.ShapeDtypeStruct((_NC, n_pad, d), jnp.float32),
        mesh=mesh,
        scratch_types=[
            pltpu.VMEM((ib, chunk), jnp.int32),
            pltpu.VMEM((ib, chunk), jnp.int32),
            [pltpu.VMEM((chunk, d), jnp.float32) for _ in range(nb)],
            [pltpu.SemaphoreType.DMA for _ in range(nb)],
            [pltpu.SemaphoreType.DMA for _ in range(nb)],
            pltpu.VMEM_SHARED((n_pad, d), jnp.float32),
        ],
    )
    def agg_kernel(hp_hbm, src_hbm, dst_hbm, z_hbm, out_hbm,
                   src_v, dst_v, bufs, gsem, ssem, acc):
        c = lax.axis_index("c")
        s = lax.axis_index("s")
        w = c * _NS + s
        pltpu.sync_copy(z_hbm, acc.at[pl.ds(s * nps, nps)])
        plsc.subcore_barrier()

        def block_body(blk, carry):
            pltpu.sync_copy(src_hbm.at[w, blk], src_v)
            pltpu.sync_copy(dst_hbm.at[w, blk], dst_v)
            for j in range(2):
                pltpu.async_copy(hp_hbm.at[src_v.at[j]], bufs[j], gsem[j])
            for j in range(ib):
                b = j % nb
                pltpu.make_async_copy(hp_hbm.at[src_v.at[j]], bufs[b],
                                      gsem[b]).wait()
                pltpu.make_async_copy(bufs[b], acc.at[dst_v.at[j]],
                                      ssem[b]).start(add=True)
                if j + 2 < ib:
                    b2 = (j + 2) % nb
                    if j >= 2:
                        pltpu.make_async_copy(bufs[b2], acc.at[dst_v.at[0]],
                                              ssem[b2]).wait()
                    pltpu.async_copy(hp_hbm.at[src_v.at[j + 2]], bufs[b2],
                                     gsem[b2])
            for j in range(ib - 2, ib):
                b = j % nb
                pltpu.make_async_copy(bufs[b], acc.at[dst_v.at[0]],
                                      ssem[b]).wait()
            return carry

        lax.fori_loop(0, nblk, block_body, 0)
        plsc.subcore_barrier()
        pltpu.sync_copy(acc.at[pl.ds(s * nps, nps)],
                        out_hbm.at[c, pl.ds(s * nps, nps)])

    return agg_kernel(hp, src4, dst4, zerosd)


def _tc_layer1(x, W1, degcol):
    n, d = x.shape
    grid = n // _BM

    def body(x_ref, w_ref, dp_ref, h_ref, hp_ref, di_ref):
        h = jnp.dot(x_ref[...], w_ref[...], preferred_element_type=jnp.float32)
        deg = dp_ref[0, :, 0:1] + dp_ref[1, :, 0:1] + 1.0
        dinv = lax.rsqrt(deg)
        h_ref[...] = h
        hp_ref[...] = h * dinv
        di_ref[...] = dinv

    return pl.pallas_call(
        body,
        grid=(grid,),
        in_specs=[
            pl.BlockSpec((_BM, d), lambda i: (i, 0)),
            pl.BlockSpec((d, d), lambda i: (0, 0)),
            pl.BlockSpec((_NC, _BM, d), lambda i: (0, i, 0)),
        ],
        out_specs=[
            pl.BlockSpec((_BM, d), lambda i: (i, 0)),
            pl.BlockSpec((_BM, d), lambda i: (i, 0)),
            pl.BlockSpec((_BM, 1), lambda i: (i, 0)),
        ],
        out_shape=[
            jax.ShapeDtypeStruct((n, d), jnp.float32),
            jax.ShapeDtypeStruct((n, d), jnp.float32),
            jax.ShapeDtypeStruct((n, 1), jnp.float32),
        ],
    )(x, W1, degcol)


def _tc_layer2(aggp, H1, dinv, b1, W2):
    n, d = H1.shape
    grid = n // _BM

    def body(a_ref, h_ref, di_ref, b_ref, w_ref, h2_ref, h2p_ref):
        di = di_ref[...]
        h1 = di * (a_ref[0] + a_ref[1]) + di * di * h_ref[...] + b_ref[...]
        h1 = jnp.maximum(h1, 0.0)
        h2 = jnp.dot(h1, w_ref[...], preferred_element_type=jnp.float32)
        h2_ref[...] = h2
        h2p_ref[...] = h2 * di

    return pl.pallas_call(
        body,
        grid=(grid,),
        in_specs=[
            pl.BlockSpec((_NC, _BM, d), lambda i: (0, i, 0)),
            pl.BlockSpec((_BM, d), lambda i: (i, 0)),
            pl.BlockSpec((_BM, 1), lambda i: (i, 0)),
            pl.BlockSpec((1, d), lambda i: (0, 0)),
            pl.BlockSpec((d, d), lambda i: (0, 0)),
        ],
        out_specs=[
            pl.BlockSpec((_BM, d), lambda i: (i, 0)),
            pl.BlockSpec((_BM, d), lambda i: (i, 0)),
        ],
        out_shape=[
            jax.ShapeDtypeStruct((n, d), jnp.float32),
            jax.ShapeDtypeStruct((n, d), jnp.float32),
        ],
    )(aggp, H1, dinv, b1, W2)


def _tc_head(aggp, H2, dinv, b2, Wh, bh, u):
    n, d = H2.shape
    a = Wh.shape[1]
    grid = n // _BM

    def body(ag_ref, h_ref, di_ref, b_ref, wh_ref, bh_ref, u_ref,
             act_ref, lp_ref):
        i = pl.program_id(0)
        di = di_ref[...]
        h2 = di * (ag_ref[0] + ag_ref[1]) + di * di * h_ref[...] + b_ref[...]
        h2 = jnp.maximum(h2, 0.0)
        logits = jnp.dot(h2, wh_ref[...], preferred_element_type=jnp.float32)
        logits = logits + bh_ref[...]
        uu = u_ref[...]
        g = -jnp.log(-jnp.log(uu + 1e-20) + 1e-20)
        z = logits + g
        zmax = jnp.max(z, axis=-1, keepdims=True)
        iota = lax.broadcasted_iota(jnp.int32, z.shape, 1)
        act = jnp.min(jnp.where(z >= zmax, iota, a), axis=-1)
        act_ref[...] = act[:, None]
        m = jnp.max(logits, axis=-1, keepdims=True)
        lse = m + jnp.log(jnp.sum(jnp.exp(logits - m), axis=-1, keepdims=True))
        sel = jnp.sum(jnp.where(iota == act[:, None], logits, 0.0),
                      axis=-1, keepdims=True)
        part = jnp.sum(sel - lse).reshape(1, 1)

        @pl.when(i == 0)
        def _():
            lp_ref[...] = part

        @pl.when(i != 0)
        def _():
            lp_ref[...] += part

    return pl.pallas_call(
        body,
        grid=(grid,),
        in_specs=[
            pl.BlockSpec((_NC, _BM, d), lambda i: (0, i, 0)),
            pl.BlockSpec((_BM, d), lambda i: (i, 0)),
            pl.BlockSpec((_BM, 1), lambda i: (i, 0)),
            pl.BlockSpec((1, d), lambda i: (0, 0)),
            pl.BlockSpec((d, a), lambda i: (0, 0)),
            pl.BlockSpec((1, a), lambda i: (0, 0)),
            pl.BlockSpec((_BM, a), lambda i: (i, 0)),
        ],
        out_specs=[
            pl.BlockSpec((_BM, 1), lambda i: (i, 0)),
            pl.BlockSpec((1, 1), lambda i: (0, 0)),
        ],
        out_shape=[
            jax.ShapeDtypeStruct((n, 1), jnp.int32),
            jax.ShapeDtypeStruct((1, 1), jnp.float32),
        ],
    )(aggp, H2, dinv, b2, Wh, bh, u)


def kernel(x, edge_index, W1, b1, W2, b2, Wh, bh):
    n, d = x.shape
    e = edge_index.shape[1]
    a = Wh.shape[1]
    nw = _NC * _NS
    n_pad = ((n + _NS * 8 - 1) // (_NS * 8)) * (_NS * 8)
    nps = n_pad // _NS

    nblk = e // (nw * 40 * _ACHUNK)
    src4 = edge_index[0].reshape(nw, nblk, 40, _ACHUNK)
    dst4 = edge_index[1].reshape(nw, nblk, 40, _ACHUNK)
    dst3d = edge_index[1].reshape(nw, e // _CHUNK // nw, _CHUNK)
    zerosd = jnp.zeros((nps, d), jnp.float32)
    onesd = jnp.ones((_CHUNK, d), jnp.float32)

    degp = _sc_degree(dst3d, zerosd, onesd, n_pad, d)
    H1, h1p, dinv = _tc_layer1(x, W1, degp)
    aggp1 = _sc_aggregate(h1p, src4, dst4, zerosd, n_pad, d)
    H2, h2p = _tc_layer2(aggp1, H1, dinv, b1.reshape(1, d), W2)
    aggp2 = _sc_aggregate(h2p, src4, dst4, zerosd, n_pad, d)
    u = jax.random.uniform(jax.random.key(42), (n, a), dtype=jnp.float32)
    act2, lp = _tc_head(aggp2, H2, dinv, b2.reshape(1, d), Wh, bh.reshape(1, a), u)
    return act2[:, 0], lp[0, 0]


# speedup vs baseline: 22.7173x; 1.0016x over previous
import functools

import jax
import jax.numpy as jnp
from jax import lax
from jax.experimental import pallas as pl
from jax.experimental.pallas import tpu as pltpu
from jax.experimental.pallas import tpu_sc as plsc

_NC = 2
_NS = 16
_CHUNK = 100
_ACHUNK = 50
_BM = 1000


def _sc_degree(dst3, zerosd, onesd, n_pad, d):
    _, rows_pw, chunk = dst3.shape
    nps = n_pad // _NS
    fk = 4
    mesh = plsc.VectorSubcoreMesh(core_axis_name="c", subcore_axis_name="s", num_cores=_NC, num_subcores=_NS)

    @functools.partial(
        pl.kernel,
        out_type=jax.ShapeDtypeStruct((_NC, n_pad, d), jnp.float32),
        mesh=mesh,
        scratch_types=[
            pltpu.VMEM((rows_pw, chunk), jnp.int32),
            pltpu.VMEM((chunk, d), jnp.float32),
            pltpu.SemaphoreType.DMA,
            pltpu.VMEM_SHARED((n_pad, d), jnp.float32),
        ],
    )
    def deg_kernel(dst_hbm, z_hbm, ones_hbm, out_hbm, dst_v, ones_v, sem, acc):
        c = lax.axis_index("c")
        s = lax.axis_index("s")
        w = c * _NS + s
        pltpu.sync_copy(z_hbm, acc.at[pl.ds(s * nps, nps)])
        pltpu.sync_copy(dst_hbm.at[w], dst_v)
        pltpu.sync_copy(ones_hbm, ones_v)
        plsc.subcore_barrier()

        def round_body(r, carry):
            for k in range(fk):
                pltpu.make_async_copy(
                    ones_v, acc.at[dst_v.at[r * fk + k]], sem).start(add=True)
            for k in range(fk):
                pltpu.make_async_copy(
                    ones_v, acc.at[dst_v.at[r * fk + k]], sem).wait()
            return carry

        lax.fori_loop(0, rows_pw // fk, round_body, 0)
        plsc.subcore_barrier()
        pltpu.sync_copy(acc.at[pl.ds(s * nps, nps)],
                        out_hbm.at[c, pl.ds(s * nps, nps)])

    return deg_kernel(dst3, zerosd, onesd)


def _sc_aggregate(hp, src4, dst4, zerosd, n_pad, d):
    _, nblk, ib, chunk = src4.shape
    nps = n_pad // _NS
    nb = 4
    mesh = plsc.VectorSubcoreMesh(core_axis_name="c", subcore_axis_name="s", num_cores=_NC, num_subcores=_NS)

    @functools.partial(
        pl.kernel,
        out_type=jax.ShapeDtypeStruct((_NC, n_pad, d), jnp.float32),
        mesh=mesh,
        scratch_types=[
            pltpu.VMEM((ib, chunk), jnp.int32),
            pltpu.VMEM((ib, chunk), jnp.int32),
            [pltpu.VMEM((chunk, d), jnp.float32) for _ in range(nb)],
            [pltpu.SemaphoreType.DMA for _ in range(nb)],
            [pltpu.SemaphoreType.DMA for _ in range(nb)],
            pltpu.VMEM_SHARED((n_pad, d), jnp.float32),
        ],
    )
    def agg_kernel(hp_hbm, src_hbm, dst_hbm, z_hbm, out_hbm,
                   src_v, dst_v, bufs, gsem, ssem, acc):
        c = lax.axis_index("c")
        s = lax.axis_index("s")
        w = c * _NS + s
        pltpu.sync_copy(z_hbm, acc.at[pl.ds(s * nps, nps)])
        plsc.subcore_barrier()

        def block_body(blk, carry):
            pltpu.sync_copy(src_hbm.at[w, blk], src_v)
            pltpu.sync_copy(dst_hbm.at[w, blk], dst_v)
            for j in range(2):
                pltpu.async_copy(hp_hbm.at[src_v.at[j]], bufs[j], gsem[j])
            for j in range(ib):
                b = j % nb
                pltpu.make_async_copy(hp_hbm.at[src_v.at[j]], bufs[b],
                                      gsem[b]).wait()
                pltpu.make_async_copy(bufs[b], acc.at[dst_v.at[j]],
                                      ssem[b]).start(add=True)
                if j + 2 < ib:
                    b2 = (j + 2) % nb
                    if j >= 2:
                        pltpu.make_async_copy(bufs[b2], acc.at[dst_v.at[0]],
                                              ssem[b2]).wait()
                    pltpu.async_copy(hp_hbm.at[src_v.at[j + 2]], bufs[b2],
                                     gsem[b2])
            for j in range(ib - 2, ib):
                b = j % nb
                pltpu.make_async_copy(bufs[b], acc.at[dst_v.at[0]],
                                      ssem[b]).wait()
            return carry

        lax.fori_loop(0, nblk, block_body, 0)
        plsc.subcore_barrier()
        pltpu.sync_copy(acc.at[pl.ds(s * nps, nps)],
                        out_hbm.at[c, pl.ds(s * nps, nps)])

    return agg_kernel(hp, src4, dst4, zerosd)


def _tc_matmul1(x, W1):
    n, d = x.shape
    grid = n // _BM

    def body(x_ref, w_ref, h_ref):
        h_ref[...] = jnp.dot(x_ref[...], w_ref[...],
                             preferred_element_type=jnp.float32)

    return pl.pallas_call(
        body,
        grid=(grid,),
        in_specs=[
            pl.BlockSpec((_BM, d), lambda i: (i, 0)),
            pl.BlockSpec((d, d), lambda i: (0, 0)),
        ],
        out_specs=pl.BlockSpec((_BM, d), lambda i: (i, 0)),
        out_shape=jax.ShapeDtypeStruct((n, d), jnp.float32),
    )(x, W1)


def _tc_scale1(H1, degp):
    n, d = H1.shape
    grid = n // _BM

    def body(h_ref, dp_ref, hp_ref, di_ref):
        deg = dp_ref[0, :, 0:1] + dp_ref[1, :, 0:1] + 1.0
        dinv = lax.rsqrt(deg)
        hp_ref[...] = h_ref[...] * dinv
        di_ref[...] = dinv

    return pl.pallas_call(
        body,
        grid=(grid,),
        in_specs=[
            pl.BlockSpec((_BM, d), lambda i: (i, 0)),
            pl.BlockSpec((_NC, _BM, d), lambda i: (0, i, 0)),
        ],
        out_specs=[
            pl.BlockSpec((_BM, d), lambda i: (i, 0)),
            pl.BlockSpec((_BM, 1), lambda i: (i, 0)),
        ],
        out_shape=[
            jax.ShapeDtypeStruct((n, d), jnp.float32),
            jax.ShapeDtypeStruct((n, 1), jnp.float32),
        ],
    )(H1, degp)


def _tc_layer2(aggp, H1, dinv, b1, W2):
    n, d = H1.shape
    grid = n // _BM

    def body(a_ref, h_ref, di_ref, b_ref, w_ref, h2_ref, h2p_ref):
        di = di_ref[...]
        h1 = di * (a_ref[0] + a_ref[1]) + di * di * h_ref[...] + b_ref[...]
        h1 = jnp.maximum(h1, 0.0)
        h2 = jnp.dot(h1, w_ref[...], preferred_element_type=jnp.float32)
        h2_ref[...] = h2
        h2p_ref[...] = h2 * di

    return pl.pallas_call(
        body,
        grid=(grid,),
        in_specs=[
            pl.BlockSpec((_NC, _BM, d), lambda i: (0, i, 0)),
            pl.BlockSpec((_BM, d), lambda i: (i, 0)),
            pl.BlockSpec((_BM, 1), lambda i: (i, 0)),
            pl.BlockSpec((1, d), lambda i: (0, 0)),
            pl.BlockSpec((d, d), lambda i: (0, 0)),
        ],
        out_specs=[
            pl.BlockSpec((_BM, d), lambda i: (i, 0)),
            pl.BlockSpec((_BM, d), lambda i: (i, 0)),
        ],
        out_shape=[
            jax.ShapeDtypeStruct((n, d), jnp.float32),
            jax.ShapeDtypeStruct((n, d), jnp.float32),
        ],
    )(aggp, H1, dinv, b1, W2)


def _tc_head(aggp, H2, dinv, b2, Wh, bh, u):
    n, d = H2.shape
    a = Wh.shape[1]
    grid = n // _BM

    def body(ag_ref, h_ref, di_ref, b_ref, wh_ref, bh_ref, u_ref,
             act_ref, lp_ref):
        i = pl.program_id(0)
        di = di_ref[...]
        h2 = di * (ag_ref[0] + ag_ref[1]) + di * di * h_ref[...] + b_ref[...]
        h2 = jnp.maximum(h2, 0.0)
        logits = jnp.dot(h2, wh_ref[...], preferred_element_type=jnp.float32)
        logits = logits + bh_ref[...]
        uu = u_ref[...]
        g = -jnp.log(-jnp.log(uu + 1e-20) + 1e-20)
        z = logits + g
        zmax = jnp.max(z, axis=-1, keepdims=True)
        iota = lax.broadcasted_iota(jnp.int32, z.shape, 1)
        act = jnp.min(jnp.where(z >= zmax, iota, a), axis=-1)
        act_ref[...] = act[:, None]
        m = jnp.max(logits, axis=-1, keepdims=True)
        lse = m + jnp.log(jnp.sum(jnp.exp(logits - m), axis=-1, keepdims=True))
        sel = jnp.sum(jnp.where(iota == act[:, None], logits, 0.0),
                      axis=-1, keepdims=True)
        part = jnp.sum(sel - lse).reshape(1, 1)

        @pl.when(i == 0)
        def _():
            lp_ref[...] = part

        @pl.when(i != 0)
        def _():
            lp_ref[...] += part

    return pl.pallas_call(
        body,
        grid=(grid,),
        in_specs=[
            pl.BlockSpec((_NC, _BM, d), lambda i: (0, i, 0)),
            pl.BlockSpec((_BM, d), lambda i: (i, 0)),
            pl.BlockSpec((_BM, 1), lambda i: (i, 0)),
            pl.BlockSpec((1, d), lambda i: (0, 0)),
            pl.BlockSpec((d, a), lambda i: (0, 0)),
            pl.BlockSpec((1, a), lambda i: (0, 0)),
            pl.BlockSpec((_BM, a), lambda i: (i, 0)),
        ],
        out_specs=[
            pl.BlockSpec((_BM, 1), lambda i: (i, 0)),
            pl.BlockSpec((1, 1), lambda i: (0, 0)),
        ],
        out_shape=[
            jax.ShapeDtypeStruct((n, 1), jnp.int32),
            jax.ShapeDtypeStruct((1, 1), jnp.float32),
        ],
    )(aggp, H2, dinv, b2, Wh, bh, u)


def kernel(x, edge_index, W1, b1, W2, b2, Wh, bh):
    n, d = x.shape
    e = edge_index.shape[1]
    a = Wh.shape[1]
    nw = _NC * _NS
    n_pad = ((n + _NS * 8 - 1) // (_NS * 8)) * (_NS * 8)
    nps = n_pad // _NS

    nblk = e // (nw * 40 * _ACHUNK)
    src4 = edge_index[0].reshape(nw, nblk, 40, _ACHUNK)
    dst4 = edge_index[1].reshape(nw, nblk, 40, _ACHUNK)
    dst3d = edge_index[1].reshape(nw, e // _CHUNK // nw, _CHUNK)
    zerosd = jnp.zeros((nps, d), jnp.float32)
    onesd = jnp.ones((_CHUNK, d), jnp.float32)

    degp = _sc_degree(dst3d, zerosd, onesd, n_pad, d)
    H1 = _tc_matmul1(x, W1)
    h1p, dinv = _tc_scale1(H1, degp)
    aggp1 = _sc_aggregate(h1p, src4, dst4, zerosd, n_pad, d)
    H2, h2p = _tc_layer2(aggp1, H1, dinv, b1.reshape(1, d), W2)
    aggp2 = _sc_aggregate(h2p, src4, dst4, zerosd, n_pad, d)
    u = jax.random.uniform(jax.random.key(42), (n, a), dtype=jnp.float32)
    act2, lp = _tc_head(aggp2, H2, dinv, b2.reshape(1, d), Wh, bh.reshape(1, a), u)
    return act2[:, 0], lp[0, 0]
